# Initial kernel scaffold; baseline (speedup 1.0000x reference)
import jax
import jax.numpy as jnp
from jax.experimental import pallas as pl


def kernel(x, embeddings):
    raise NotImplementedError("write your pallas kernel here")


# speedup vs baseline: 1.6072x; 1.6072x over previous
import functools

import jax
import jax.numpy as jnp
from jax import lax
from jax.experimental import pallas as pl
from jax.experimental.pallas import tpu as pltpu
from jax.experimental.pallas import tpu_sc as plsc

M_BLK = 1024
N_BLK = 2048

_SC_CORES = 2
_SC_SUBCORES = 16
_SC_WORKERS = _SC_CORES * _SC_SUBCORES
_GATHER_CHUNK = 128


def _argmin_body(x_ref, e_ref, idx_ref, min_s, arg_s):
    c = pl.program_id(1)
    nc = pl.num_programs(1)
    x = x_ref[...]
    e = e_ref[...]
    sim = jnp.dot(x, e, preferred_element_type=jnp.float32)
    x2 = jnp.sum(x * x, axis=1, keepdims=True)
    e2 = jnp.sum(e * e, axis=0, keepdims=True)
    d = (x2 + e2) - 2.0 * sim
    m = jnp.min(d, axis=1, keepdims=True)
    ii = lax.broadcasted_iota(jnp.int32, d.shape, 1)
    a = jnp.min(jnp.where(d == m, ii, d.shape[1]), axis=1, keepdims=True)
    a = a + c * N_BLK

    @pl.when(c == 0)
    def _():
        min_s[...] = m.astype(jnp.bfloat16)
        arg_s[...] = a

    @pl.when(c > 0)
    def _():
        prev = min_s[...].astype(jnp.float32)
        better = m < prev
        min_s[...] = jnp.where(better, m, prev).astype(jnp.bfloat16)
        arg_s[...] = jnp.where(better, a, arg_s[...])

    @pl.when(c == nc - 1)
    def _():
        idx_ref[...] = arg_s[...]


def _argmin_call(flat, embeddings):
    m, k = flat.shape
    n = embeddings.shape[1]
    grid = (m // M_BLK, n // N_BLK)
    return pl.pallas_call(
        _argmin_body,
        grid=grid,
        in_specs=[
            pl.BlockSpec((M_BLK, k), lambda r, c: (r, 0)),
            pl.BlockSpec((k, N_BLK), lambda r, c: (0, c)),
        ],
        out_specs=pl.BlockSpec((M_BLK, 1), lambda r, c: (r, 0)),
        out_shape=jax.ShapeDtypeStruct((m, 1), jnp.int32),
        scratch_shapes=[
            pltpu.VMEM((M_BLK, 1), jnp.bfloat16),
            pltpu.VMEM((M_BLK, 1), jnp.int32),
        ],
    )(flat, embeddings)


def _gather_rows(table, idx):
    n, d = table.shape
    b = idx.shape[0]
    bpw = b // _SC_WORKERS
    mesh = plsc.VectorSubcoreMesh(core_axis_name="c", subcore_axis_name="s")

    @functools.partial(
        pl.kernel,
        mesh=mesh,
        out_type=jax.ShapeDtypeStruct((b, d), table.dtype),
        scratch_types=[
            pltpu.VMEM((_GATHER_CHUNK,), jnp.int32),
            pltpu.VMEM((_GATHER_CHUNK, d), jnp.float32),
            pltpu.SemaphoreType.DMA,
        ],
    )
    def k(table_hbm, idx_hbm, out_hbm, idx_v, rows_v, sem):
        wid = lax.axis_index("s") * _SC_CORES + lax.axis_index("c")
        base = wid * bpw
        for ci in range(bpw // _GATHER_CHUNK):
            off = base + ci * _GATHER_CHUNK
            pltpu.sync_copy(idx_hbm.at[pl.ds(off, _GATHER_CHUNK)], idx_v)
            pltpu.async_copy(table_hbm.at[idx_v], rows_v, sem).wait()
            pltpu.sync_copy(rows_v, out_hbm.at[pl.ds(off, _GATHER_CHUNK)])

    return k(table, idx)


def kernel(x, embeddings):
    input_shape = x.shape
    k = embeddings.shape[0]
    flat = x.reshape(-1, k)
    idx = _argmin_call(flat, embeddings).reshape(-1)
    table = embeddings.T
    quantized = _gather_rows(table, idx)
    return quantized.reshape(input_shape)
